# baseline (speedup 1.0000x reference)
import jax
import jax.numpy as jnp
from jax import lax
from jax.experimental import pallas as pl
from jax.experimental.pallas import tpu as pltpu
from jax.experimental.pallas import tpu_sc as plsc

NUM_CORES = 2
NUM_SUBCORES = 16
NUM_WORKERS = NUM_CORES * NUM_SUBCORES
BATCH = 16384
HIDDEN = 128
B_PER_W = BATCH // NUM_WORKERS
CHUNK = 128
N_CHUNKS = B_PER_W // CHUNK


def _gather_body(labels_hbm, table_hbm, out_hbm, idx_v, rows_v, lsems, gsems, wsem):
    wid = lax.axis_index("s") * NUM_CORES + lax.axis_index("c")
    base = wid * B_PER_W
    label_copies = [
        pltpu.async_copy(
            labels_hbm.at[pl.ds(base + c * CHUNK, CHUNK)],
            idx_v.at[pl.ds(c * CHUNK, CHUNK)],
            lsems.at[c],
        )
        for c in range(N_CHUNKS)
    ]
    gathers = []
    for c in range(N_CHUNKS):
        label_copies[c].wait()
        gathers.append(
            pltpu.async_copy(
                table_hbm.at[idx_v.at[pl.ds(c * CHUNK, CHUNK)]],
                rows_v.at[pl.ds(c * CHUNK, CHUNK)],
                gsems.at[c],
            )
        )
    writes = []
    for c in range(N_CHUNKS):
        gathers[c].wait()
        writes.append(
            pltpu.async_copy(
                rows_v.at[pl.ds(c * CHUNK, CHUNK)],
                out_hbm.at[pl.ds(base + c * CHUNK, CHUNK)],
                wsem,
            )
        )
    for w in writes:
        w.wait()


@jax.jit
def kernel(labels, embedding_table):
    mesh = plsc.VectorSubcoreMesh(core_axis_name="c", subcore_axis_name="s")
    f = pl.kernel(
        _gather_body,
        out_type=jax.ShapeDtypeStruct((BATCH, HIDDEN), jnp.float32),
        mesh=mesh,
        scratch_types=[
            pltpu.VMEM((B_PER_W,), jnp.int32),
            pltpu.VMEM((B_PER_W, HIDDEN), jnp.float32),
            pltpu.SemaphoreType.DMA((N_CHUNKS,)),
            pltpu.SemaphoreType.DMA((N_CHUNKS,)),
            pltpu.SemaphoreType.DMA,
        ],
    )
    return f(labels.astype(jnp.int32), embedding_table)


# speedup vs baseline: 1.0054x; 1.0054x over previous
import jax
import jax.numpy as jnp
from jax import lax
from jax.experimental import pallas as pl
from jax.experimental.pallas import tpu as pltpu
from jax.experimental.pallas import tpu_sc as plsc

NUM_CORES = 2
NUM_SUBCORES = 16
NUM_WORKERS = NUM_CORES * NUM_SUBCORES
BATCH = 16384
HIDDEN = 128
B_PER_W = BATCH // NUM_WORKERS
CHUNKS = (64, 192, 192, 64)
N_CHUNKS = len(CHUNKS)
OFFSETS = tuple(sum(CHUNKS[:i]) for i in range(N_CHUNKS))


def _gather_body(labels_hbm, table_hbm, out_hbm, idx_v, rows_v, gsems, wsem):
    wid = lax.axis_index("s") * NUM_CORES + lax.axis_index("c")
    base = wid * B_PER_W
    pltpu.sync_copy(labels_hbm.at[pl.ds(base, B_PER_W)], idx_v)
    gathers = [
        pltpu.async_copy(
            table_hbm.at[idx_v.at[pl.ds(OFFSETS[c], CHUNKS[c])]],
            rows_v.at[pl.ds(OFFSETS[c], CHUNKS[c])],
            gsems.at[c],
        )
        for c in range(N_CHUNKS)
    ]
    writes = []
    for c in range(N_CHUNKS):
        gathers[c].wait()
        writes.append(
            pltpu.async_copy(
                rows_v.at[pl.ds(OFFSETS[c], CHUNKS[c])],
                out_hbm.at[pl.ds(base + OFFSETS[c], CHUNKS[c])],
                wsem,
            )
        )
    for w in writes:
        w.wait()


@jax.jit
def kernel(labels, embedding_table):
    mesh = plsc.VectorSubcoreMesh(core_axis_name="c", subcore_axis_name="s")
    f = pl.kernel(
        _gather_body,
        out_type=jax.ShapeDtypeStruct((BATCH, HIDDEN), jnp.float32),
        mesh=mesh,
        scratch_types=[
            pltpu.VMEM((B_PER_W,), jnp.int32),
            pltpu.VMEM((B_PER_W, HIDDEN), jnp.float32),
            pltpu.SemaphoreType.DMA((N_CHUNKS,)),
            pltpu.SemaphoreType.DMA,
        ],
    )
    return f(labels.astype(jnp.int32), embedding_table)


# speedup vs baseline: 1.0092x; 1.0038x over previous
import jax
import jax.numpy as jnp
from jax import lax
from jax.experimental import pallas as pl
from jax.experimental.pallas import tpu as pltpu
from jax.experimental.pallas import tpu_sc as plsc

NUM_CORES = 2
NUM_SUBCORES = 16
NUM_WORKERS = NUM_CORES * NUM_SUBCORES
BATCH = 16384
HIDDEN = 128
B_PER_W = BATCH // NUM_WORKERS
CHUNKS = (128, 128, 128, 128)
N_CHUNKS = len(CHUNKS)
OFFSETS = tuple(sum(CHUNKS[:i]) for i in range(N_CHUNKS))


def _gather_body(labels_hbm, table_hbm, out_hbm, idx_v, rows_v, gsems, wsem):
    wid = lax.axis_index("s") * NUM_CORES + lax.axis_index("c")
    base = wid * B_PER_W
    pltpu.sync_copy(labels_hbm.at[pl.ds(base, B_PER_W)], idx_v)
    gathers = [
        pltpu.async_copy(
            table_hbm.at[idx_v.at[pl.ds(OFFSETS[c], CHUNKS[c])]],
            rows_v.at[pl.ds(OFFSETS[c], CHUNKS[c])],
            gsems.at[c],
        )
        for c in range(N_CHUNKS)
    ]
    writes = []
    for c in range(N_CHUNKS):
        gathers[c].wait()
        writes.append(
            pltpu.async_copy(
                rows_v.at[pl.ds(OFFSETS[c], CHUNKS[c])],
                out_hbm.at[pl.ds(base + OFFSETS[c], CHUNKS[c])],
                wsem,
            )
        )
    for w in writes:
        w.wait()


@jax.jit
def kernel(labels, embedding_table):
    mesh = plsc.VectorSubcoreMesh(core_axis_name="c", subcore_axis_name="s")
    f = pl.kernel(
        _gather_body,
        out_type=jax.ShapeDtypeStruct((BATCH, HIDDEN), jnp.float32),
        mesh=mesh,
        scratch_types=[
            pltpu.VMEM((B_PER_W,), jnp.int32),
            pltpu.VMEM((B_PER_W, HIDDEN), jnp.float32),
            pltpu.SemaphoreType.DMA((N_CHUNKS,)),
            pltpu.SemaphoreType.DMA,
        ],
    )
    return f(labels.astype(jnp.int32), embedding_table)
